# Initial kernel scaffold; baseline (speedup 1.0000x reference)
import jax
import jax.numpy as jnp
from jax.experimental import pallas as pl


def kernel(inputs, weight):
    raise NotImplementedError("write your pallas kernel here")


# speedup vs baseline: 2.1310x; 2.1310x over previous
import functools

import jax
import jax.numpy as jnp
from jax import lax
from jax.experimental import pallas as pl
from jax.experimental.pallas import tpu as pltpu
from jax.experimental.pallas import tpu_sc as plsc

_N = 18432
_K = 1024
_D = 64
_BLK = 1024
_COMMIT = 0.25


def _tc_body(x_ref, w_ref, idx_ref, loss_ref):
    i = pl.program_id(0)
    x = x_ref[...]
    w = w_ref[...]
    mm = lax.dot_general(x, w, (((1,), (1,)), ((), ())),
                         preferred_element_type=jnp.float32)
    z2 = jnp.sum(x * x, axis=1, keepdims=True)
    w2 = jnp.sum(w * w, axis=1)
    dist = (z2 + w2[None, :]) - 2.0 * mm
    minval = jnp.min(dist, axis=1, keepdims=True)
    kiota = lax.broadcasted_iota(jnp.int32, dist.shape, 1)
    idx = jnp.min(jnp.where(dist == minval, kiota, _K), axis=1, keepdims=True)
    idx_ref[...] = idx

    @pl.when(i == 0)
    def _init():
        loss_ref[...] = jnp.zeros_like(loss_ref)

    loss_ref[...] = loss_ref[...] + jnp.sum(minval)


_tc_call = pl.pallas_call(
    _tc_body,
    grid=(_N // _BLK,),
    in_specs=[
        pl.BlockSpec((_BLK, _D), lambda i: (i, 0)),
        pl.BlockSpec((_K, _D), lambda i: (0, 0)),
    ],
    out_specs=[
        pl.BlockSpec((_BLK, 1), lambda i: (i, 0)),
        pl.BlockSpec((1, 1), lambda i: (0, 0)),
    ],
    out_shape=[
        jax.ShapeDtypeStruct((_N, 1), jnp.int32),
        jax.ShapeDtypeStruct((1, 1), jnp.float32),
    ],
)


_DP = 128


@functools.cache
def _make_sc_gather():
    info = plsc.get_sparse_core_info()
    nc, ns = info.num_cores, info.num_subcores
    nw = nc * ns
    bpw = _N // nw
    ch = 96
    n_ch = bpw // ch
    assert bpw % ch == 0 and _N % nw == 0
    mesh = plsc.VectorSubcoreMesh(core_axis_name="c", subcore_axis_name="s")

    @functools.partial(
        pl.kernel,
        mesh=mesh,
        out_type=jax.ShapeDtypeStruct((_N, _DP), jnp.float32),
        scratch_types=[
            pltpu.VMEM((bpw,), jnp.int32),
            pltpu.VMEM((bpw, _DP), jnp.float32),
            pltpu.SemaphoreType.DMA,
        ],
    )
    def gather_k(idx_hbm, table_hbm, out_hbm, idx_v, rows_v, sem):
        wid = lax.axis_index("s") * nc + lax.axis_index("c")
        base = wid * bpw
        pltpu.sync_copy(idx_hbm.at[pl.ds(base, bpw)], idx_v)
        copies = []
        for j in range(n_ch):
            copies.append(pltpu.async_copy(
                table_hbm.at[idx_v.at[pl.ds(j * ch, ch)]],
                rows_v.at[pl.ds(j * ch, ch)],
                sem,
            ))
        for cp in copies:
            cp.wait()
        pltpu.sync_copy(rows_v, out_hbm.at[pl.ds(base, bpw)])

    return gather_k


def kernel(inputs, weight):
    idx2, loss_sum = _tc_call(inputs, weight)
    gather_k = _make_sc_gather()
    idx_flat = idx2.reshape(_N)
    table_pad = jnp.pad(weight, ((0, 0), (0, _DP - _D)))
    quant_pad = gather_k(idx_flat, table_pad)
    quantized = quant_pad[:, :_D]
    s = loss_sum[0, 0]
    q_latent_loss = s / (_N * _D)
    e_term = _COMMIT * q_latent_loss
    return quantized, q_latent_loss, e_term, idx2


# speedup vs baseline: 2.6262x; 1.2324x over previous
import functools

import jax
import jax.numpy as jnp
from jax import lax
from jax.experimental import pallas as pl
from jax.experimental.pallas import tpu as pltpu
from jax.experimental.pallas import tpu_sc as plsc

_N = 18432
_K = 1024
_D = 64
_BLK = 1024
_COMMIT = 0.25


def _tc_body(xt_ref, wt_ref, w2_ref, idx_ref, loss_ref):
    i = pl.program_id(0)
    xt = xt_ref[...]
    wt = wt_ref[...]
    mmt = lax.dot_general(wt, xt, (((0,), (0,)), ((), ())),
                          preferred_element_type=jnp.float32)
    z2 = jnp.sum(xt * xt, axis=0, keepdims=True)
    w2 = w2_ref[...]
    dist = (z2 + w2) - 2.0 * mmt
    minval = jnp.min(dist, axis=0, keepdims=True)
    kiota = lax.broadcasted_iota(jnp.int32, dist.shape, 0)
    idx = jnp.min(jnp.where(dist == minval, kiota, _K), axis=0, keepdims=True)
    idx_ref[...] = idx[None]

    @pl.when(i == 0)
    def _init():
        loss_ref[...] = jnp.zeros_like(loss_ref)

    loss_ref[...] = loss_ref[...] + jnp.sum(minval)


_tc_call = pl.pallas_call(
    _tc_body,
    grid=(_N // _BLK,),
    in_specs=[
        pl.BlockSpec((_D, _BLK), lambda i: (0, i)),
        pl.BlockSpec((_D, _K), lambda i: (0, 0)),
        pl.BlockSpec((_K, 1), lambda i: (0, 0)),
    ],
    out_specs=[
        pl.BlockSpec((1, 1, _BLK), lambda i: (i, 0, 0)),
        pl.BlockSpec((1, 1), lambda i: (0, 0)),
    ],
    out_shape=[
        jax.ShapeDtypeStruct((_N // _BLK, 1, _BLK), jnp.int32),
        jax.ShapeDtypeStruct((1, 1), jnp.float32),
    ],
)


_DP = 128


@functools.cache
def _make_sc_gather():
    info = plsc.get_sparse_core_info()
    nc, ns = info.num_cores, info.num_subcores
    nw = nc * ns
    bpw = _N // nw
    ch = 96
    n_ch = bpw // ch
    assert bpw % ch == 0 and _N % nw == 0
    mesh = plsc.VectorSubcoreMesh(core_axis_name="c", subcore_axis_name="s")

    @functools.partial(
        pl.kernel,
        mesh=mesh,
        out_type=jax.ShapeDtypeStruct((_N, _DP), jnp.float32),
        scratch_types=[
            pltpu.VMEM((bpw,), jnp.int32),
            pltpu.VMEM((bpw, _DP), jnp.float32),
            pltpu.SemaphoreType.DMA,
        ],
    )
    def gather_k(idx_hbm, table_hbm, out_hbm, idx_v, rows_v, sem):
        wid = lax.axis_index("s") * nc + lax.axis_index("c")
        base = wid * bpw
        pltpu.sync_copy(idx_hbm.at[pl.ds(base, bpw)], idx_v)
        copies = []
        for j in range(n_ch):
            copies.append(pltpu.async_copy(
                table_hbm.at[idx_v.at[pl.ds(j * ch, ch)]],
                rows_v.at[pl.ds(j * ch, ch)],
                sem,
            ))
        for cp in copies:
            cp.wait()
        pltpu.sync_copy(rows_v, out_hbm.at[pl.ds(base, bpw)])

    return gather_k


def kernel(inputs, weight):
    xt = inputs.T
    wt = weight.T
    w2col = jnp.sum(weight * weight, axis=1, keepdims=True)
    idx3, loss_sum = _tc_call(xt, wt, w2col)
    idx_flat = idx3.reshape(_N)
    gather_k = _make_sc_gather()
    table_pad = jnp.pad(weight, ((0, 0), (0, _DP - _D)))
    quant_pad = gather_k(idx_flat, table_pad)
    quantized = quant_pad[:, :_D]
    s = loss_sum[0, 0]
    q_latent_loss = s / (_N * _D)
    e_term = _COMMIT * q_latent_loss
    return quantized, q_latent_loss, e_term, idx_flat.reshape(_N, 1)


# speedup vs baseline: 2.8569x; 1.0879x over previous
import functools

import jax
import jax.numpy as jnp
from jax import lax
from jax.experimental import pallas as pl
from jax.experimental.pallas import tpu as pltpu
from jax.experimental.pallas import tpu_sc as plsc

_N = 18432
_K = 1024
_D = 64
_BLK = 1024
_COMMIT = 0.25


def _tc_body(xt_ref, wt_ref, w2_ref, idx_ref, loss_ref):
    i = pl.program_id(0)
    xt = xt_ref[...]
    wt = wt_ref[...]
    mmt = lax.dot_general(wt, xt, (((0,), (0,)), ((), ())),
                          preferred_element_type=jnp.float32)
    z2 = jnp.sum(xt * xt, axis=0, keepdims=True)
    w2 = w2_ref[...]
    dist = (z2 + w2) - mmt
    minval = jnp.min(dist, axis=0, keepdims=True)
    idx = jnp.argmin(dist, axis=0).reshape(1, _BLK)
    idx_ref[...] = idx[None]

    @pl.when(i == 0)
    def _init():
        loss_ref[...] = jnp.zeros_like(loss_ref)

    loss_ref[...] = loss_ref[...] + jnp.sum(minval)


_tc_call = pl.pallas_call(
    _tc_body,
    grid=(_N // _BLK,),
    in_specs=[
        pl.BlockSpec((_D, _BLK), lambda i: (0, i)),
        pl.BlockSpec((_D, _K), lambda i: (0, 0)),
        pl.BlockSpec((_K, 1), lambda i: (0, 0)),
    ],
    out_specs=[
        pl.BlockSpec((1, 1, _BLK), lambda i: (i, 0, 0)),
        pl.BlockSpec((1, 1), lambda i: (0, 0)),
    ],
    out_shape=[
        jax.ShapeDtypeStruct((_N // _BLK, 1, _BLK), jnp.int32),
        jax.ShapeDtypeStruct((1, 1), jnp.float32),
    ],
)


_DP = 128


@functools.cache
def _make_sc_gather():
    info = plsc.get_sparse_core_info()
    nc, ns = info.num_cores, info.num_subcores
    nw = nc * ns
    bpw = _N // nw
    ch = 96
    n_ch = bpw // ch
    assert bpw % ch == 0 and _N % nw == 0
    mesh = plsc.VectorSubcoreMesh(core_axis_name="c", subcore_axis_name="s")

    @functools.partial(
        pl.kernel,
        mesh=mesh,
        out_type=jax.ShapeDtypeStruct((_N, _DP), jnp.float32),
        scratch_types=[
            pltpu.VMEM((bpw,), jnp.int32),
            pltpu.VMEM((bpw, _DP), jnp.float32),
            pltpu.SemaphoreType.DMA,
        ],
    )
    def gather_k(idx_hbm, table_hbm, out_hbm, idx_v, rows_v, sem):
        wid = lax.axis_index("s") * nc + lax.axis_index("c")
        base = wid * bpw
        pltpu.sync_copy(idx_hbm.at[pl.ds(base, bpw)], idx_v)
        copies = []
        for j in range(n_ch):
            copies.append(pltpu.async_copy(
                table_hbm.at[idx_v.at[pl.ds(j * ch, ch)]],
                rows_v.at[pl.ds(j * ch, ch)],
                sem,
            ))
        for cp in copies:
            cp.wait()
        pltpu.sync_copy(rows_v, out_hbm.at[pl.ds(base, bpw)])

    return gather_k


def kernel(inputs, weight):
    xt = inputs.T
    wt = weight.T + weight.T
    w2col = jnp.sum(weight * weight, axis=1, keepdims=True)
    idx3, loss_sum = _tc_call(xt, wt, w2col)
    idx_flat = idx3.reshape(_N)
    gather_k = _make_sc_gather()
    table_pad = jnp.pad(weight, ((0, 0), (0, _DP - _D)))
    quant_pad = gather_k(idx_flat, table_pad)
    quantized = quant_pad[:, :_D]
    s = loss_sum[0, 0]
    q_latent_loss = s / (_N * _D)
    e_term = _COMMIT * q_latent_loss
    return quantized, q_latent_loss, e_term, idx_flat.reshape(_N, 1)
